# baseline (speedup 1.0000x reference)
import functools

import jax
import jax.numpy as jnp
from jax import lax
from jax.experimental import pallas as pl
from jax.experimental.pallas import tpu as pltpu
from jax.experimental.pallas import tpu_sc as plsc


@functools.lru_cache(maxsize=None)
def _build_pos_kernel(d, em, h, w):
    info = plsc.get_sparse_core_info()
    NC, NS, L = info.num_cores, info.num_subcores, info.num_lanes
    NW = NC * NS
    f2 = 2 * em
    assert f2 % NW == 0 and w % L == 0
    CPW = f2 // NW
    plane = h * w
    blk = CPW * plane
    mesh = plsc.VectorSubcoreMesh(core_axis_name="c", subcore_axis_name="s")

    @functools.partial(
        pl.kernel,
        mesh=mesh,
        compiler_params=pltpu.CompilerParams(needs_layout_passes=False),
        out_type=jax.ShapeDtypeStruct((d * f2 * plane,), jnp.float32),
        scratch_types=[
            pltpu.VMEM((CPW, w), jnp.float32),
            pltpu.VMEM((blk,), jnp.float32),
            pltpu.SemaphoreType.DMA,
        ],
    )
    def pos_kernel(tbl_hbm, out_hbm, tbl_v, plane_v, sem):
        wid = lax.axis_index("s") * NC + lax.axis_index("c")
        pltpu.sync_copy(tbl_hbm.at[pl.ds(wid * CPW, CPW)], tbl_v)

        is_col = (wid * CPW) < em
        iota = lax.iota(jnp.int32, L)

        @pl.when(is_col)
        def _():
            for c in range(CPW):
                for j in range(w // L):
                    v = tbl_v[c, pl.ds(j * L, L)]

                    def ybody(y, carry, c=c, j=j, v=v):
                        plane_v[pl.ds(c * plane + y * w + j * L, L)] = v
                        return carry

                    lax.fori_loop(0, h, ybody, 0)

        @pl.when(jnp.logical_not(is_col))
        def _():
            for c in range(CPW):

                def ybody(y, carry, c=c):
                    base = (y // L) * L
                    vy = tbl_v[c, pl.ds(base, L)]
                    s = jnp.sum(jnp.where(iota == (y - base), vy, 0.0))
                    vfull = jnp.full((L,), s, jnp.float32)
                    for j in range(w // L):
                        plane_v[pl.ds(c * plane + y * w + j * L, L)] = vfull
                    return carry

                lax.fori_loop(0, h, ybody, 0)

        base = wid * blk
        for g in range(0, d, 8):
            handles = [
                pltpu.async_copy(
                    plane_v, out_hbm.at[pl.ds(dd * f2 * plane + base, blk)], sem)
                for dd in range(g, min(g + 8, d))
            ]
            for hd in handles:
                hd.wait()

    return pos_kernel


def kernel(scan, row_embed, col_embed, dep_embed):
    d, em, h, w = scan.shape
    tbl = jnp.concatenate(
        [col_embed.T[:, :w], row_embed.T[:, :h]], axis=0)
    out_flat = _build_pos_kernel(d, em, h, w)(tbl)
    return out_flat.reshape(d, 2 * em, h, w)


# speedup vs baseline: 1.3270x; 1.3270x over previous
import functools

import jax
import jax.numpy as jnp
from jax import lax
from jax.experimental import pallas as pl
from jax.experimental.pallas import tpu as pltpu
from jax.experimental.pallas import tpu_sc as plsc


@functools.lru_cache(maxsize=None)
def _build_pos_kernel(d, em, h, w):
    info = plsc.get_sparse_core_info()
    NC, NS, L = info.num_cores, info.num_subcores, info.num_lanes
    NW = NC * NS
    f2 = 2 * em
    assert f2 % NW == 0 and w % L == 0
    CPW = f2 // NW
    plane = h * w
    blk = CPW * plane
    mesh = plsc.VectorSubcoreMesh(core_axis_name="c", subcore_axis_name="s")

    @functools.partial(
        pl.kernel,
        mesh=mesh,
        compiler_params=pltpu.CompilerParams(needs_layout_passes=False),
        out_type=jax.ShapeDtypeStruct((d, f2, h, w), jnp.float32),
        scratch_types=[
            pltpu.VMEM((CPW, w), jnp.float32),
            pltpu.VMEM((CPW, h, w), jnp.float32),
            pltpu.SemaphoreType.DMA,
        ],
    )
    def pos_kernel(tbl_hbm, out_hbm, tbl_v, plane_v, sem):
        wid = lax.axis_index("s") * NC + lax.axis_index("c")
        pltpu.sync_copy(tbl_hbm.at[pl.ds(wid * CPW, CPW)], tbl_v)

        is_col = (wid * CPW) < em
        iota = lax.iota(jnp.int32, L)

        @pl.when(is_col)
        def _():
            for c in range(CPW):
                for j in range(w // L):
                    v = tbl_v[c, pl.ds(j * L, L)]

                    def ybody(y, carry, c=c, j=j, v=v):
                        plane_v[c, y, pl.ds(j * L, L)] = v
                        return carry

                    lax.fori_loop(0, h, ybody, 0)

        @pl.when(jnp.logical_not(is_col))
        def _():
            for c in range(CPW):

                def ybody(y, carry, c=c):
                    base = (y // L) * L
                    vy = tbl_v[c, pl.ds(base, L)]
                    s = jnp.sum(jnp.where(iota == (y - base), vy, 0.0))
                    vfull = jnp.full((L,), s, jnp.float32)
                    for j in range(w // L):
                        plane_v[c, y, pl.ds(j * L, L)] = vfull
                    return carry

                lax.fori_loop(0, h, ybody, 0)

        c0 = wid * CPW
        for g in range(0, d, 8):
            handles = [
                pltpu.async_copy(
                    plane_v, out_hbm.at[dd, pl.ds(c0, CPW)], sem)
                for dd in range(g, min(g + 8, d))
            ]
            for hd in handles:
                hd.wait()

    return pos_kernel


def kernel(scan, row_embed, col_embed, dep_embed):
    d, em, h, w = scan.shape
    tbl = jnp.concatenate(
        [col_embed.T[:, :w], row_embed.T[:, :h]], axis=0)
    return _build_pos_kernel(d, em, h, w)(tbl)


# speedup vs baseline: 2.4803x; 1.8691x over previous
import functools

import jax
import jax.numpy as jnp
from jax import lax
from jax.experimental import pallas as pl
from jax.experimental.pallas import tpu as pltpu
from jax.experimental.pallas import tpu_sc as plsc


@functools.lru_cache(maxsize=None)
def _build_pos_kernel(d, em, h, w):
    info = plsc.get_sparse_core_info()
    NC, NS, L = info.num_cores, info.num_subcores, info.num_lanes
    NW = NC * NS
    f2 = 2 * em
    NG = NW // 2
    YPG = h // NG
    DPW = d // 2
    assert h % NG == 0 and d % 2 == 0 and em % L == 0
    row_w = w * f2
    stripe = YPG * row_w
    mesh = plsc.VectorSubcoreMesh(core_axis_name="c", subcore_axis_name="s")

    @functools.partial(
        pl.kernel,
        mesh=mesh,
        compiler_params=pltpu.CompilerParams(needs_layout_passes=False),
        out_type=jax.ShapeDtypeStruct((d * h * row_w,), jnp.float32),
        scratch_types=[
            pltpu.VMEM((h * em,), jnp.float32),
            pltpu.VMEM((YPG * em,), jnp.float32),
            pltpu.VMEM((stripe,), jnp.float32),
            pltpu.SemaphoreType.DMA,
        ],
    )
    def pos_kernel(col_hbm, row_hbm, out_hbm, col_v, row_v, stripe_v, sem):
        wid = lax.axis_index("s") * NC + lax.axis_index("c")
        g = wid // 2
        half = wid % 2
        pltpu.sync_copy(col_hbm.at[pl.ds(0, w * em)], col_v)
        pltpu.sync_copy(row_hbm.at[pl.ds(g * YPG * em, YPG * em)], row_v)

        rvs = [[row_v[pl.ds(yy * em + k * L, L)] for k in range(em // L)]
               for yy in range(YPG)]

        def xbody(x, carry):
            for k in range(em // L):
                v = col_v[pl.ds(x * em + k * L, L)]
                for yy in range(YPG):
                    stripe_v[pl.ds(yy * row_w + x * f2 + k * L, L)] = v
            for yy in range(YPG):
                for k in range(em // L):
                    stripe_v[pl.ds(yy * row_w + x * f2 + em + k * L, L)] = (
                        rvs[yy][k])
            return carry

        lax.fori_loop(0, w, xbody, 0)

        base = g * stripe
        for gg in range(0, DPW, 8):
            handles = [
                pltpu.async_copy(
                    stripe_v,
                    out_hbm.at[pl.ds((half * DPW + dd) * h * row_w + base,
                                     stripe)],
                    sem)
                for dd in range(gg, min(gg + 8, DPW))
            ]
            for hd in handles:
                hd.wait()

    return pos_kernel


def kernel(scan, row_embed, col_embed, dep_embed):
    d, em, h, w = scan.shape
    col_flat = col_embed[:w].reshape(-1)
    row_flat = row_embed[:h].reshape(-1)
    out_flat = _build_pos_kernel(d, em, h, w)(col_flat, row_flat)
    return out_flat.reshape(d, h, w, 2 * em).transpose(0, 3, 1, 2)


# speedup vs baseline: 7.3945x; 2.9812x over previous
import functools

import jax
import jax.numpy as jnp
from jax import lax
from jax.experimental import pallas as pl
from jax.experimental.pallas import tpu as pltpu
from jax.experimental.pallas import tpu_sc as plsc


@functools.lru_cache(maxsize=None)
def _build_pos_kernel(d, em, h, w):
    info = plsc.get_sparse_core_info()
    NC, NS, L = info.num_cores, info.num_subcores, info.num_lanes
    NW = NC * NS
    f2 = 2 * em
    NG = NW // 2
    YPG = h // NG
    DPW = d // 2
    assert h % NG == 0 and d % 2 == 0 and em % L == 0
    mesh = plsc.VectorSubcoreMesh(core_axis_name="c", subcore_axis_name="s")

    @functools.partial(
        pl.kernel,
        mesh=mesh,
        compiler_params=pltpu.CompilerParams(needs_layout_passes=False),
        out_type=jax.ShapeDtypeStruct((d, h, w, f2), jnp.float32),
        scratch_types=[
            pltpu.VMEM((w, em), jnp.float32),
            pltpu.VMEM((h, em), jnp.float32),
            pltpu.VMEM((YPG, w, f2), jnp.float32),
            pltpu.SemaphoreType.DMA,
        ],
    )
    def pos_kernel(col_hbm, row_hbm, out_hbm, col_v, row_v, stripe_v, sem):
        wid = lax.axis_index("s") * NC + lax.axis_index("c")
        g = wid // 2
        half = wid % 2
        pltpu.sync_copy(col_hbm, col_v)
        pltpu.sync_copy(row_hbm, row_v)

        rvs = [[row_v[g * YPG + yy, pl.ds(k * L, L)] for k in range(em // L)]
               for yy in range(YPG)]

        def xbody(x, carry):
            for k in range(em // L):
                v = col_v[x, pl.ds(k * L, L)]
                for yy in range(YPG):
                    stripe_v[yy, x, pl.ds(k * L, L)] = v
            for yy in range(YPG):
                for k in range(em // L):
                    stripe_v[yy, x, pl.ds(em + k * L, L)] = rvs[yy][k]
            return carry

        lax.fori_loop(0, w, xbody, 0)

        y0 = g * YPG
        for gg in range(0, DPW, 8):
            handles = [
                pltpu.async_copy(
                    stripe_v,
                    out_hbm.at[half * DPW + dd, pl.ds(y0, YPG)],
                    sem)
                for dd in range(gg, min(gg + 8, DPW))
            ]
            for hd in handles:
                hd.wait()

    return pos_kernel


def kernel(scan, row_embed, col_embed, dep_embed):
    d, em, h, w = scan.shape
    out = _build_pos_kernel(d, em, h, w)(col_embed[:w], row_embed[:h])
    return out.transpose(0, 3, 1, 2)
